# Initial kernel scaffold; baseline (speedup 1.0000x reference)
import jax
import jax.numpy as jnp
from jax.experimental import pallas as pl


def kernel(hash_ids, table, pos):
    raise NotImplementedError("write your pallas kernel here")


# speedup vs baseline: 3.9213x; 3.9213x over previous
import functools

import jax
import jax.numpy as jnp
from jax import lax
from jax.experimental import pallas as pl
from jax.experimental.pallas import tpu as pltpu
from jax.experimental.pallas import tpu_sc as plsc

VOCAB = 100000
EMB = 32
NBLOCKS = 100
BATCH = 4096


def _sc_body(hash_hbm, table_hbm, pos_hbm, out_hbm, idx_v, pos_v, pos_sh, buf_v, sem):
    info = plsc.get_sparse_core_info()
    nc = info.num_cores
    nw = nc * info.num_subcores
    rows_per_w = BATCH // nw

    sid = lax.axis_index("s")
    wid = sid * nc + lax.axis_index("c")
    base = wid * rows_per_w

    pltpu.sync_copy(hash_hbm.at[pl.ds(base, rows_per_w)], idx_v)

    @pl.when(sid == 0)
    def _():
        pltpu.sync_copy(pos_hbm, pos_v)
        pltpu.sync_copy(pos_v, pos_sh)

    plsc.subcore_barrier()

    def step(r, carry):
        pltpu.sync_copy(pos_sh, buf_v)
        pltpu.async_copy(table_hbm.at[idx_v.at[r]], buf_v, sem, add=True).wait()
        pltpu.sync_copy(buf_v, out_hbm.at[base + r])
        return carry

    lax.fori_loop(0, rows_per_w, step, 0)


def kernel(hash_ids, table, pos):
    pos2d = pos.reshape(NBLOCKS, EMB)
    mesh = plsc.VectorSubcoreMesh(core_axis_name="c", subcore_axis_name="s")
    info = plsc.get_sparse_core_info()
    rows_per_w = BATCH // (info.num_cores * info.num_subcores)

    run = pl.kernel(
        _sc_body,
        out_type=jax.ShapeDtypeStruct((BATCH, NBLOCKS, EMB), jnp.float32),
        mesh=mesh,
        scratch_types=[
            pltpu.VMEM((rows_per_w, NBLOCKS), jnp.int32),
            pltpu.VMEM((NBLOCKS, EMB), jnp.float32),
            pltpu.VMEM_SHARED((NBLOCKS, EMB), jnp.float32),
            pltpu.VMEM((NBLOCKS, EMB), jnp.float32),
            pltpu.SemaphoreType.DMA,
        ],
        compiler_params=pltpu.CompilerParams(use_tc_tiling_on_sc=False),
    )
    return run(hash_ids, table, pos2d)
